# speedup vs baseline: 6.3530x; 6.3530x over previous
import jax
import jax.numpy as jnp
from jax import lax
from jax.experimental import pallas as pl
from jax.experimental.pallas import tpu as pltpu
from jax.experimental.pallas import tpu_sc as plsc

NC = 2
NS = 16
NW = NC * NS
CHUNK = 128


def _gather_body(table_hbm, idx_hbm, out_hbm, idx_v, rows_v, gsem):
    n_chunk_rows = idx_hbm.shape[0]
    rows_per_w = n_chunk_rows // NW
    wid = lax.axis_index("s") * NC + lax.axis_index("c")
    base = wid * rows_per_w

    pltpu.sync_copy(idx_hbm.at[pl.ds(base, rows_per_w)], idx_v)

    @pl.loop(0, rows_per_w)
    def _(j):
        pltpu.async_copy(table_hbm.at[idx_v.at[j]], rows_v, gsem).wait()
        pltpu.sync_copy(rows_v, out_hbm.at[pl.ds((base + j) * CHUNK, CHUNK)])


def kernel(x, table):
    B, S = x.shape
    V, D = table.shape
    total = B * S
    idx2d = x.reshape(total // CHUNK, CHUNK)

    mesh = plsc.VectorSubcoreMesh(
        core_axis_name="c", subcore_axis_name="s",
        num_cores=NC, num_subcores=NS)

    run = pl.kernel(
        _gather_body,
        out_type=jax.ShapeDtypeStruct((total, D), jnp.float32),
        mesh=mesh,
        scratch_types=[
            pltpu.VMEM((total // CHUNK // NW, CHUNK), jnp.int32),
            pltpu.VMEM((CHUNK, D), jnp.float32),
            pltpu.SemaphoreType.DMA,
        ],
    )
    out = run(table, idx2d)
    return out.reshape(B, S, D)


# speedup vs baseline: 9.1456x; 1.4396x over previous
import jax
import jax.numpy as jnp
from jax import lax
from jax.experimental import pallas as pl
from jax.experimental.pallas import tpu as pltpu
from jax.experimental.pallas import tpu_sc as plsc

NC = 2
NS = 16
NW = NC * NS
CHUNK = 128


NBUF = 4


def _gather_body(table_hbm, idx_hbm, out_hbm, idx_v,
                 rows0, rows1, rows2, rows3,
                 g0, g1, g2, g3, w0, w1, w2, w3):
    rows = (rows0, rows1, rows2, rows3)
    gsem = (g0, g1, g2, g3)
    wsem = (w0, w1, w2, w3)

    n_chunk_rows = idx_hbm.shape[0]
    rows_per_w = n_chunk_rows // NW
    ngroups = rows_per_w // NBUF
    wid = lax.axis_index("s") * NC + lax.axis_index("c")
    base = wid * rows_per_w

    pltpu.sync_copy(idx_hbm.at[pl.ds(base, rows_per_w)], idx_v)

    def drain_gather(j, b):
        pltpu.make_async_copy(table_hbm.at[idx_v.at[j]], rows[b], gsem[b]).wait()

    def drain_write(b):
        pltpu.make_async_copy(rows[b], out_hbm.at[pl.ds(0, CHUNK)], wsem[b]).wait()

    def fire_gather(j, b):
        pltpu.async_copy(table_hbm.at[idx_v.at[j]], rows[b], gsem[b])

    def fire_write(j, b):
        pltpu.async_copy(rows[b], out_hbm.at[pl.ds((base + j) * CHUNK, CHUNK)],
                         wsem[b])

    for b in range(NBUF):
        fire_gather(b, b)

    @pl.loop(1, ngroups)
    def _(g):
        jprev = (g - 1) * NBUF
        jcur = g * NBUF
        for b in range(NBUF):
            drain_gather(jprev + b, b)
            fire_write(jprev + b, b)
        for b in range(NBUF):
            drain_write(b)
            fire_gather(jcur + b, b)

    jlast = (ngroups - 1) * NBUF
    for b in range(NBUF):
        drain_gather(jlast + b, b)
        fire_write(jlast + b, b)
    for b in range(NBUF):
        drain_write(b)


def kernel(x, table):
    B, S = x.shape
    V, D = table.shape
    total = B * S
    idx2d = x.reshape(total // CHUNK, CHUNK)

    mesh = plsc.VectorSubcoreMesh(
        core_axis_name="c", subcore_axis_name="s",
        num_cores=NC, num_subcores=NS)

    run = pl.kernel(
        _gather_body,
        out_type=jax.ShapeDtypeStruct((total, D), jnp.float32),
        mesh=mesh,
        scratch_types=[
            pltpu.VMEM((total // CHUNK // NW, CHUNK), jnp.int32),
        ] + [pltpu.VMEM((CHUNK, D), jnp.float32) for _ in range(NBUF)]
          + [pltpu.SemaphoreType.DMA for _ in range(2 * NBUF)],
    )
    out = run(table, idx2d)
    return out.reshape(B, S, D)


# speedup vs baseline: 9.1782x; 1.0036x over previous
import jax
import jax.numpy as jnp
from jax import lax
from jax.experimental import pallas as pl
from jax.experimental.pallas import tpu as pltpu
from jax.experimental.pallas import tpu_sc as plsc

NC = 2
NS = 16
NW = NC * NS
CHUNK = 128


NBUF = 4


def _gather_body(table_hbm, idx_hbm, out_hbm, idx_v,
                 rows0, rows1, rows2, rows3,
                 g0, g1, g2, g3, w0, w1, w2, w3):
    rows = (rows0, rows1, rows2, rows3)
    gsem = (g0, g1, g2, g3)
    wsem = (w0, w1, w2, w3)

    n_chunk_rows = idx_hbm.shape[0]
    rows_per_w = n_chunk_rows // NW
    ngroups = rows_per_w // NBUF
    wid = lax.axis_index("s") * NC + lax.axis_index("c")
    base = wid * rows_per_w

    pltpu.sync_copy(idx_hbm.at[pl.ds(base, rows_per_w)], idx_v)

    def drain_gather(j, b):
        pltpu.make_async_copy(table_hbm.at[idx_v.at[j]], rows[b], gsem[b]).wait()

    def drain_write(b):
        pltpu.make_async_copy(rows[b], out_hbm.at[pl.ds(0, CHUNK)], wsem[b]).wait()

    def fire_gather(j, b):
        pltpu.async_copy(table_hbm.at[idx_v.at[j]], rows[b], gsem[b])

    def fire_write(j, b):
        pltpu.async_copy(rows[b], out_hbm.at[pl.ds((base + j) * CHUNK, CHUNK)],
                         wsem[b])

    M = NBUF // 2
    niter = (rows_per_w // M)

    def bset(g):
        return [(g % 2) * M + i for i in range(M)]

    for i in range(M):
        fire_gather(i, bset(0)[i])
    for i in range(M):
        drain_gather(i, bset(0)[i])
        fire_write(i, bset(0)[i])
    for i in range(M):
        fire_gather(M + i, bset(1)[i])

    @pl.loop(2, niter, step=2)
    def _(g):
        for h in range(2):
            jprev = (g + h - 1) * M
            jcur = (g + h) * M
            for i, b in enumerate(bset(h - 1)):
                drain_gather(jprev + i, b)
                fire_write(jprev + i, b)
            for i, b in enumerate(bset(h)):
                drain_write(b)
                fire_gather(jcur + i, b)

    jlast = (niter - 1) * M
    for i, b in enumerate(bset(niter - 1)):
        drain_gather(jlast + i, b)
        fire_write(jlast + i, b)
    for b in bset(niter - 2) + bset(niter - 1):
        drain_write(b)


def kernel(x, table):
    B, S = x.shape
    V, D = table.shape
    total = B * S
    idx2d = x.reshape(total // CHUNK, CHUNK)

    mesh = plsc.VectorSubcoreMesh(
        core_axis_name="c", subcore_axis_name="s",
        num_cores=NC, num_subcores=NS)

    run = pl.kernel(
        _gather_body,
        out_type=jax.ShapeDtypeStruct((total, D), jnp.float32),
        mesh=mesh,
        scratch_types=[
            pltpu.VMEM((total // CHUNK // NW, CHUNK), jnp.int32),
        ] + [pltpu.VMEM((CHUNK, D), jnp.float32) for _ in range(NBUF)]
          + [pltpu.SemaphoreType.DMA for _ in range(2 * NBUF)],
    )
    out = run(table, idx2d)
    return out.reshape(B, S, D)


# speedup vs baseline: 9.2159x; 1.0041x over previous
import jax
import jax.numpy as jnp
from jax import lax
from jax.experimental import pallas as pl
from jax.experimental.pallas import tpu as pltpu
from jax.experimental.pallas import tpu_sc as plsc

NC = 2
NS = 16
NW = NC * NS
CHUNK = 128
CPG = 2
GPW_ROWS = CHUNK * CPG


def _gather_body(table_hbm, idx_hbm, out_hbm, idx_v,
                 rows0, rows1, g0, g1, w0, w1):
    rows = (rows0, rows1)
    gsem = (g0, g1)
    wsem = (w0, w1)

    n_chunk_rows = idx_hbm.shape[0]
    chunks_per_w = n_chunk_rows // NW
    ngroups = chunks_per_w // CPG
    wid = lax.axis_index("s") * NC + lax.axis_index("c")
    base_chunk = wid * chunks_per_w

    pltpu.sync_copy(idx_hbm.at[pl.ds(base_chunk, chunks_per_w)], idx_v)

    def fire_gathers(g, b):
        for h in range(CPG):
            pltpu.async_copy(
                table_hbm.at[idx_v.at[g * CPG + h]],
                rows[b].at[pl.ds(h * CHUNK, CHUNK)], gsem[b])

    def drain_gathers(g, b):
        for h in range(CPG):
            pltpu.make_async_copy(
                table_hbm.at[idx_v.at[g * CPG + h]],
                rows[b].at[pl.ds(h * CHUNK, CHUNK)], gsem[b]).wait()

    def fire_write(g, b):
        pltpu.async_copy(
            rows[b],
            out_hbm.at[pl.ds((base_chunk + g * CPG) * CHUNK, GPW_ROWS)],
            wsem[b])

    def drain_write(b):
        pltpu.make_async_copy(
            rows[b], out_hbm.at[pl.ds(0, GPW_ROWS)], wsem[b]).wait()

    fire_gathers(0, 0)
    drain_gathers(0, 0)
    fire_write(0, 0)
    fire_gathers(1, 1)

    @pl.loop(2, ngroups, step=2)
    def _(g):
        for h in range(2):
            b_prev = (h + 1) % 2
            b_cur = h
            drain_gathers(g + h - 1, b_prev)
            fire_write(g + h - 1, b_prev)
            drain_write(b_cur)
            fire_gathers(g + h, b_cur)

    drain_gathers(ngroups - 1, (ngroups - 1) % 2)
    fire_write(ngroups - 1, (ngroups - 1) % 2)
    drain_write(0)
    drain_write(1)


def kernel(x, table):
    B, S = x.shape
    V, D = table.shape
    total = B * S
    idx2d = x.reshape(total // CHUNK, CHUNK)

    mesh = plsc.VectorSubcoreMesh(
        core_axis_name="c", subcore_axis_name="s",
        num_cores=NC, num_subcores=NS)

    run = pl.kernel(
        _gather_body,
        out_type=jax.ShapeDtypeStruct((total, D), jnp.float32),
        mesh=mesh,
        scratch_types=[
            pltpu.VMEM((total // CHUNK // NW, CHUNK), jnp.int32),
            pltpu.VMEM((GPW_ROWS, D), jnp.float32),
            pltpu.VMEM((GPW_ROWS, D), jnp.float32),
            pltpu.SemaphoreType.DMA,
            pltpu.SemaphoreType.DMA,
            pltpu.SemaphoreType.DMA,
            pltpu.SemaphoreType.DMA,
        ],
    )
    out = run(table, idx2d)
    return out.reshape(B, S, D)
